# baseline (speedup 1.0000x reference)
import functools

import jax
import jax.numpy as jnp
from jax.experimental import pallas as pl
from jax.experimental.pallas import tpu as pltpu


def _dot(a, b):
    return jnp.dot(a, b, preferred_element_type=jnp.float32)


def _node_proj_kernel(x_ref, wnbr_ref, bnbr_ref, wself_ref, bself_ref,
                      wi_ref, bi_ref, wj_ref, bj_ref,
                      t_ref, sf_ref, addi_ref, addj_ref):
    x = x_ref[...]
    t_ref[...] = _dot(x, wnbr_ref[...]) + bnbr_ref[...]
    sf_ref[...] = _dot(x, wself_ref[...]) + bself_ref[...]
    addi_ref[...] = _dot(x, wi_ref[...]) + bi_ref[...]
    addj_ref[...] = _dot(x, wj_ref[...]) + bj_ref[...]


def _edge_kernel(ef_ref, adj_ref, ind_ref, addi_ref, addj_ref, t_ref, sf_ref,
                 we1_ref, be1_ref, we2_ref, be2_ref, we3_ref, be3_ref,
                 wag_ref, wblk_ref, bblk_ref, wa3_ref, ba3_ref,
                 wc1a_ref, wc1b_ref, bc1_ref, wc2_ref, bc2_ref,
                 out_ref, *, ti, n, cout):
    bf16 = jnp.bfloat16
    ef = ef_ref[0]
    e = ef.shape[1]

    we1b = jnp.broadcast_to(we1_ref[...][None], (ti, e, 64))
    pe = jax.lax.dot_general(ef, we1b, (((1,), (1,)), ((0,), (0,))),
                             preferred_element_type=jnp.float32)
    pe = pe.reshape(ti * n, 64).astype(bf16)
    pe = jnp.maximum(pe + be1_ref[...], 0)
    pe = jnp.maximum(_dot(pe, we2_ref[...]).astype(bf16) + be2_ref[...], 0)
    pe = jnp.maximum(_dot(pe, we3_ref[...]).astype(bf16) + be3_ref[...], 0)

    pe_aug = jnp.concatenate([pe, ind_ref[...]], axis=1)
    w_aug = jnp.concatenate([wag_ref[...], addi_ref[0].astype(bf16)], axis=0)
    ag = _dot(pe_aug, w_aug).reshape(ti, n, 128).astype(bf16)
    ag = jnp.maximum(ag + addj_ref[0][None, :, :], 0).reshape(ti * n, 128)

    hg = _dot(ag, wblk_ref[...]).astype(bf16) + bblk_ref[...]
    gate = jax.nn.sigmoid(hg[:, :cout])
    h2 = jnp.maximum(hg[:, cout:], 0)

    logits = (_dot(h2, wa3_ref[...]) + ba3_ref[...]).reshape(ti, n)
    mask = adj_ref[0] > 0.0
    ml = jnp.where(mask, logits, -3.0e38)
    mx = jnp.max(ml, axis=1, keepdims=True)
    mxc = jnp.where(mx > -1.0e37, mx, 0.0)
    ew = jnp.where(mask, jnp.exp(logits - mxc), 0.0)
    rdenom = 1.0 / jnp.maximum(jnp.sum(ew, axis=1, keepdims=True), 1e-30)

    gt = gate.reshape(ti, n, cout) * t_ref[0][None]
    prod = gt * ew.astype(bf16).reshape(ti, n, 1)
    msg = jnp.sum(prod.astype(jnp.float32), axis=1) * rdenom

    hid = jnp.maximum(_dot(sf_ref[0], wc1a_ref[...]) + _dot(msg, wc1b_ref[...])
                      + bc1_ref[...], 0.0)
    out_ref[0] = _dot(hid, wc2_ref[...]) + bc2_ref[...]


def kernel(x, adjacency, edge_features, W_self, b_self, W_nbr, b_nbr,
           We1, be1, We2, be2, We3, be3, Wa1, ba1, Wa2, ba2, Wa3, ba3,
           Wg1, bg1, Wg2, bg2, Wc1, bc1, Wc2, bc2):
    B, N, C = x.shape
    Cout = W_self.shape[1]
    E = edge_features.shape[-1]
    TI = 16
    f32 = jnp.float32
    bf16 = jnp.bfloat16

    W_i = jnp.concatenate([Wa1[:C], jnp.zeros((C, 64), f32)], axis=1)
    b_i = jnp.concatenate([ba1, jnp.zeros((64,), f32)])
    W_j = jnp.concatenate([Wa1[C:2 * C], Wg1[:C]], axis=1)
    b_j = jnp.concatenate([jnp.zeros((64,), f32), bg1])

    xf = x.reshape(B * N, C)
    row = lambda v: v.reshape(1, -1)
    full = lambda a: pl.BlockSpec(a.shape, lambda: tuple(0 for _ in a.shape))
    node_ins = (xf, W_nbr, row(b_nbr), W_self, row(b_self),
                W_i, row(b_i), W_j, row(b_j))
    t, sf, addi, addj = pl.pallas_call(
        _node_proj_kernel,
        grid=(),
        in_specs=[full(a) for a in node_ins],
        out_specs=[pl.BlockSpec((B * N, Cout), lambda: (0, 0)),
                   pl.BlockSpec((B * N, Cout), lambda: (0, 0)),
                   pl.BlockSpec((B * N, 128), lambda: (0, 0)),
                   pl.BlockSpec((B * N, 128), lambda: (0, 0))],
        out_shape=[jax.ShapeDtypeStruct((B * N, Cout), f32),
                   jax.ShapeDtypeStruct((B * N, Cout), f32),
                   jax.ShapeDtypeStruct((B * N, 128), f32),
                   jax.ShapeDtypeStruct((B * N, 128), f32)],
    )(*node_ins)
    t = t.reshape(B, N, Cout)
    sf = sf.reshape(B, N, Cout)
    addi = addi.reshape(B, N, 128)
    addj = addj.reshape(B, N, 128)

    efT = jnp.transpose(edge_features.astype(bf16), (0, 1, 3, 2))

    ind = (jnp.arange(TI * N, dtype=jnp.int32)[:, None] // N
           == jnp.arange(TI, dtype=jnp.int32)[None, :]).astype(bf16)

    W_ag = jnp.concatenate([Wa1[2 * C:], Wg1[C:]], axis=1).astype(bf16)
    W_blk = jnp.concatenate([
        jnp.concatenate([jnp.zeros((64, Cout), f32), Wa2], axis=1),
        jnp.concatenate([Wg2, jnp.zeros((64, 32), f32)], axis=1)],
        axis=0).astype(bf16)
    b_blk = jnp.concatenate([bg2, ba2]).reshape(1, Cout + 32).astype(bf16)

    wspec = lambda a: pl.BlockSpec(a.shape, lambda b, i: tuple(0 for _ in a.shape))
    weight_ins = (We1.astype(bf16), row(be1).astype(bf16),
                  We2.astype(bf16), row(be2).astype(bf16),
                  We3.astype(bf16), row(be3).astype(bf16),
                  W_ag, W_blk, b_blk,
                  Wa3.astype(bf16), row(ba3),
                  Wc1[:Cout], Wc1[Cout:], row(bc1), Wc2, row(bc2))

    out = pl.pallas_call(
        functools.partial(_edge_kernel, ti=TI, n=N, cout=Cout),
        grid=(B, N // TI),
        in_specs=[
            pl.BlockSpec((1, TI, E, N), lambda b, i: (b, i, 0, 0)),
            pl.BlockSpec((1, TI, N), lambda b, i: (b, i, 0)),
            pl.BlockSpec((TI * N, TI), lambda b, i: (0, 0)),
            pl.BlockSpec((1, TI, 128), lambda b, i: (b, i, 0)),
            pl.BlockSpec((1, N, 128), lambda b, i: (b, 0, 0)),
            pl.BlockSpec((1, N, Cout), lambda b, i: (b, 0, 0)),
            pl.BlockSpec((1, TI, Cout), lambda b, i: (b, i, 0)),
        ] + [wspec(a) for a in weight_ins],
        out_specs=pl.BlockSpec((1, TI, Cout), lambda b, i: (b, i, 0)),
        out_shape=jax.ShapeDtypeStruct((B, N, Cout), f32),
        compiler_params=pltpu.CompilerParams(
            dimension_semantics=("parallel", "parallel")),
    )(efT, adjacency, ind, addi, addj.astype(bf16),
      t.astype(bf16), sf, *weight_ins)
    return out


# speedup vs baseline: 1.0523x; 1.0523x over previous
import functools

import jax
import jax.numpy as jnp
from jax.experimental import pallas as pl
from jax.experimental.pallas import tpu as pltpu


def _dot(a, b):
    return jnp.dot(a, b, preferred_element_type=jnp.float32)


def _node_proj_kernel(x_ref, wnbr_ref, bnbr_ref, wself_ref, bself_ref,
                      wi_ref, bi_ref, wj_ref, bj_ref,
                      t_ref, sf_ref, addi_ref, addj_ref):
    x = x_ref[...]
    t_ref[...] = _dot(x, wnbr_ref[...]) + bnbr_ref[...]
    sf_ref[...] = _dot(x, wself_ref[...]) + bself_ref[...]
    addi_ref[...] = _dot(x, wi_ref[...]) + bi_ref[...]
    addj_ref[...] = _dot(x, wj_ref[...]) + bj_ref[...]


def _edge_kernel(ef_ref, adj_ref, ind_ref, addi_ref, addj_ref, t_ref, sf_ref,
                 we1_ref, be1_ref, we2_ref, be2_ref, we3_ref, be3_ref,
                 wag_ref, wblk_ref, bblk_ref, wa3_ref, ba3_ref,
                 wc1a_ref, wc1b_ref, bc1_ref, wc2_ref, bc2_ref,
                 out_ref, *, ti, n, cout):
    bf16 = jnp.bfloat16
    ef = ef_ref[0]
    e = ef.shape[1]

    we1b = jnp.broadcast_to(we1_ref[...][None], (ti, e, 64))
    pe = jax.lax.dot_general(ef, we1b, (((1,), (1,)), ((0,), (0,))),
                             preferred_element_type=jnp.float32)
    pe = pe.reshape(ti * n, 64).astype(bf16)
    pe = jnp.maximum(pe + be1_ref[...], 0)
    pe = jnp.maximum(_dot(pe, we2_ref[...]).astype(bf16) + be2_ref[...], 0)
    pe = jnp.maximum(_dot(pe, we3_ref[...]).astype(bf16) + be3_ref[...], 0)

    pe_aug = jnp.concatenate([pe, ind_ref[...]], axis=1)
    w_aug = jnp.concatenate([wag_ref[...], addi_ref[0].astype(bf16)], axis=0)
    ag = _dot(pe_aug, w_aug).reshape(ti, n, 128).astype(bf16)
    ag = jnp.maximum(ag + addj_ref[0][None, :, :], 0).reshape(ti * n, 128)

    hg = _dot(ag, wblk_ref[...]).astype(bf16) + bblk_ref[...]
    gate = jax.nn.sigmoid(hg[:, :cout])
    h2 = jnp.maximum(hg[:, cout:], 0)

    logits = (_dot(h2, wa3_ref[...]) + ba3_ref[...]).reshape(ti, n)
    mask = adj_ref[0] > 0.0
    ml = jnp.where(mask, logits, -3.0e38)
    mx = jnp.max(ml, axis=1, keepdims=True)
    mxc = jnp.where(mx > -1.0e37, mx, 0.0)
    ew = jnp.where(mask, jnp.exp(logits - mxc), 0.0)
    rdenom = 1.0 / jnp.maximum(jnp.sum(ew, axis=1, keepdims=True), 1e-30)

    gt = gate.reshape(ti, n, cout) * t_ref[0][None]
    prod = gt * ew.astype(bf16).reshape(ti, n, 1)
    msg = jnp.sum(prod.astype(jnp.float32), axis=1) * rdenom

    hid = jnp.maximum(_dot(sf_ref[0], wc1a_ref[...]) + _dot(msg, wc1b_ref[...])
                      + bc1_ref[...], 0.0)
    out_ref[0] = _dot(hid, wc2_ref[...]) + bc2_ref[...]


def kernel(x, adjacency, edge_features, W_self, b_self, W_nbr, b_nbr,
           We1, be1, We2, be2, We3, be3, Wa1, ba1, Wa2, ba2, Wa3, ba3,
           Wg1, bg1, Wg2, bg2, Wc1, bc1, Wc2, bc2):
    B, N, C = x.shape
    Cout = W_self.shape[1]
    E = edge_features.shape[-1]
    TI = 32
    f32 = jnp.float32
    bf16 = jnp.bfloat16

    W_i = jnp.concatenate([Wa1[:C], jnp.zeros((C, 64), f32)], axis=1)
    b_i = jnp.concatenate([ba1, jnp.zeros((64,), f32)])
    W_j = jnp.concatenate([Wa1[C:2 * C], Wg1[:C]], axis=1)
    b_j = jnp.concatenate([jnp.zeros((64,), f32), bg1])

    xf = x.reshape(B * N, C)
    row = lambda v: v.reshape(1, -1)
    full = lambda a: pl.BlockSpec(a.shape, lambda: tuple(0 for _ in a.shape))
    node_ins = (xf, W_nbr, row(b_nbr), W_self, row(b_self),
                W_i, row(b_i), W_j, row(b_j))
    t, sf, addi, addj = pl.pallas_call(
        _node_proj_kernel,
        grid=(),
        in_specs=[full(a) for a in node_ins],
        out_specs=[pl.BlockSpec((B * N, Cout), lambda: (0, 0)),
                   pl.BlockSpec((B * N, Cout), lambda: (0, 0)),
                   pl.BlockSpec((B * N, 128), lambda: (0, 0)),
                   pl.BlockSpec((B * N, 128), lambda: (0, 0))],
        out_shape=[jax.ShapeDtypeStruct((B * N, Cout), f32),
                   jax.ShapeDtypeStruct((B * N, Cout), f32),
                   jax.ShapeDtypeStruct((B * N, 128), f32),
                   jax.ShapeDtypeStruct((B * N, 128), f32)],
    )(*node_ins)
    t = t.reshape(B, N, Cout)
    sf = sf.reshape(B, N, Cout)
    addi = addi.reshape(B, N, 128)
    addj = addj.reshape(B, N, 128)

    efT = jnp.transpose(edge_features.astype(bf16), (0, 1, 3, 2))

    ind = (jnp.arange(TI * N, dtype=jnp.int32)[:, None] // N
           == jnp.arange(TI, dtype=jnp.int32)[None, :]).astype(bf16)

    W_ag = jnp.concatenate([Wa1[2 * C:], Wg1[C:]], axis=1).astype(bf16)
    W_blk = jnp.concatenate([
        jnp.concatenate([jnp.zeros((64, Cout), f32), Wa2], axis=1),
        jnp.concatenate([Wg2, jnp.zeros((64, 32), f32)], axis=1)],
        axis=0).astype(bf16)
    b_blk = jnp.concatenate([bg2, ba2]).reshape(1, Cout + 32).astype(bf16)

    wspec = lambda a: pl.BlockSpec(a.shape, lambda b, i: tuple(0 for _ in a.shape))
    weight_ins = (We1.astype(bf16), row(be1).astype(bf16),
                  We2.astype(bf16), row(be2).astype(bf16),
                  We3.astype(bf16), row(be3).astype(bf16),
                  W_ag, W_blk, b_blk,
                  Wa3.astype(bf16), row(ba3),
                  Wc1[:Cout], Wc1[Cout:], row(bc1), Wc2, row(bc2))

    out = pl.pallas_call(
        functools.partial(_edge_kernel, ti=TI, n=N, cout=Cout),
        grid=(B, N // TI),
        in_specs=[
            pl.BlockSpec((1, TI, E, N), lambda b, i: (b, i, 0, 0)),
            pl.BlockSpec((1, TI, N), lambda b, i: (b, i, 0)),
            pl.BlockSpec((TI * N, TI), lambda b, i: (0, 0)),
            pl.BlockSpec((1, TI, 128), lambda b, i: (b, i, 0)),
            pl.BlockSpec((1, N, 128), lambda b, i: (b, 0, 0)),
            pl.BlockSpec((1, N, Cout), lambda b, i: (b, 0, 0)),
            pl.BlockSpec((1, TI, Cout), lambda b, i: (b, i, 0)),
        ] + [wspec(a) for a in weight_ins],
        out_specs=pl.BlockSpec((1, TI, Cout), lambda b, i: (b, i, 0)),
        out_shape=jax.ShapeDtypeStruct((B, N, Cout), f32),
        compiler_params=pltpu.CompilerParams(
            dimension_semantics=("parallel", "parallel")),
    )(efT, adjacency, ind, addi, addj.astype(bf16),
      t.astype(bf16), sf, *weight_ins)
    return out


# speedup vs baseline: 1.0654x; 1.0125x over previous
import functools

import jax
import jax.numpy as jnp
from jax.experimental import pallas as pl
from jax.experimental.pallas import tpu as pltpu


def _dot(a, b):
    return jnp.dot(a, b, preferred_element_type=jnp.float32)


def _node_proj_kernel(x_ref, wnbr_ref, bnbr_ref, wself_ref, bself_ref,
                      wi_ref, bi_ref, wj_ref, bj_ref,
                      t_ref, sf_ref, addi_ref, addj_ref):
    x = x_ref[...]
    t_ref[...] = _dot(x, wnbr_ref[...]) + bnbr_ref[...]
    sf_ref[...] = _dot(x, wself_ref[...]) + bself_ref[...]
    addi_ref[...] = _dot(x, wi_ref[...]) + bi_ref[...]
    addj_ref[...] = _dot(x, wj_ref[...]) + bj_ref[...]


def _edge_kernel(ef_ref, adj_ref, ind_ref, addi_ref, addj_ref, t_ref, sf_ref,
                 we1_ref, be1_ref, we2_ref, be2_ref, we3_ref, be3_ref,
                 wag_ref, wblk_ref, bblk_ref, wa3_ref, ba3_ref,
                 wc1a_ref, wc1b_ref, bc1_ref, wc2_ref, bc2_ref,
                 out_ref, *, ti, n, cout):
    bf16 = jnp.bfloat16
    ef = ef_ref[0]
    e = ef.shape[1]

    we1b = jnp.broadcast_to(we1_ref[...][None], (ti, e, 64))
    pe = jax.lax.dot_general(ef, we1b, (((1,), (1,)), ((0,), (0,))),
                             preferred_element_type=jnp.float32)
    pe = pe.reshape(ti * n, 64).astype(bf16)
    pe = jnp.maximum(pe + be1_ref[...], 0)
    pe = jnp.maximum(_dot(pe, we2_ref[...]).astype(bf16) + be2_ref[...], 0)
    pe = jnp.maximum(_dot(pe, we3_ref[...]).astype(bf16) + be3_ref[...], 0)

    pe_aug = jnp.concatenate([pe, ind_ref[...]], axis=1)
    w_aug = jnp.concatenate([wag_ref[...], addi_ref[0].astype(bf16)], axis=0)
    ag = _dot(pe_aug, w_aug).reshape(ti, n, 128).astype(bf16)
    ag = jnp.maximum(ag + addj_ref[0][None, :, :], 0).reshape(ti * n, 128)

    hg = _dot(ag, wblk_ref[...]).astype(bf16) + bblk_ref[...]
    gate = jax.nn.sigmoid(hg[:, :cout])
    h2 = jnp.maximum(hg[:, cout:], 0)

    logits = (_dot(h2, wa3_ref[...]) + ba3_ref[...]).reshape(ti, n)
    mask = adj_ref[0] > 0.0
    ml = jnp.where(mask, logits, -3.0e38)
    mx = jnp.max(ml, axis=1, keepdims=True)
    mxc = jnp.where(mx > -1.0e37, mx, 0.0)
    ew = jnp.where(mask, jnp.exp(logits - mxc), 0.0)
    rdenom = 1.0 / jnp.maximum(jnp.sum(ew, axis=1, keepdims=True), 1e-30)

    gt = gate.reshape(ti, n, cout) * t_ref[0][None]
    prod = gt * ew.astype(bf16).reshape(ti, n, 1)
    msg = jnp.sum(prod.astype(jnp.float32), axis=1) * rdenom

    hid = jnp.maximum(_dot(sf_ref[0], wc1a_ref[...]) + _dot(msg, wc1b_ref[...])
                      + bc1_ref[...], 0.0)
    out_ref[0] = _dot(hid, wc2_ref[...]) + bc2_ref[...]


def kernel(x, adjacency, edge_features, W_self, b_self, W_nbr, b_nbr,
           We1, be1, We2, be2, We3, be3, Wa1, ba1, Wa2, ba2, Wa3, ba3,
           Wg1, bg1, Wg2, bg2, Wc1, bc1, Wc2, bc2):
    B, N, C = x.shape
    Cout = W_self.shape[1]
    E = edge_features.shape[-1]
    TI = 64
    f32 = jnp.float32
    bf16 = jnp.bfloat16

    W_i = jnp.concatenate([Wa1[:C], jnp.zeros((C, 64), f32)], axis=1)
    b_i = jnp.concatenate([ba1, jnp.zeros((64,), f32)])
    W_j = jnp.concatenate([Wa1[C:2 * C], Wg1[:C]], axis=1)
    b_j = jnp.concatenate([jnp.zeros((64,), f32), bg1])

    xf = x.reshape(B * N, C)
    row = lambda v: v.reshape(1, -1)
    full = lambda a: pl.BlockSpec(a.shape, lambda: tuple(0 for _ in a.shape))
    node_ins = (xf, W_nbr, row(b_nbr), W_self, row(b_self),
                W_i, row(b_i), W_j, row(b_j))
    t, sf, addi, addj = pl.pallas_call(
        _node_proj_kernel,
        grid=(),
        in_specs=[full(a) for a in node_ins],
        out_specs=[pl.BlockSpec((B * N, Cout), lambda: (0, 0)),
                   pl.BlockSpec((B * N, Cout), lambda: (0, 0)),
                   pl.BlockSpec((B * N, 128), lambda: (0, 0)),
                   pl.BlockSpec((B * N, 128), lambda: (0, 0))],
        out_shape=[jax.ShapeDtypeStruct((B * N, Cout), f32),
                   jax.ShapeDtypeStruct((B * N, Cout), f32),
                   jax.ShapeDtypeStruct((B * N, 128), f32),
                   jax.ShapeDtypeStruct((B * N, 128), f32)],
    )(*node_ins)
    t = t.reshape(B, N, Cout)
    sf = sf.reshape(B, N, Cout)
    addi = addi.reshape(B, N, 128)
    addj = addj.reshape(B, N, 128)

    efT = jnp.transpose(edge_features.astype(bf16), (0, 1, 3, 2))

    ind = (jnp.arange(TI * N, dtype=jnp.int32)[:, None] // N
           == jnp.arange(TI, dtype=jnp.int32)[None, :]).astype(bf16)

    W_ag = jnp.concatenate([Wa1[2 * C:], Wg1[C:]], axis=1).astype(bf16)
    W_blk = jnp.concatenate([
        jnp.concatenate([jnp.zeros((64, Cout), f32), Wa2], axis=1),
        jnp.concatenate([Wg2, jnp.zeros((64, 32), f32)], axis=1)],
        axis=0).astype(bf16)
    b_blk = jnp.concatenate([bg2, ba2]).reshape(1, Cout + 32).astype(bf16)

    wspec = lambda a: pl.BlockSpec(a.shape, lambda b, i: tuple(0 for _ in a.shape))
    weight_ins = (We1.astype(bf16), row(be1).astype(bf16),
                  We2.astype(bf16), row(be2).astype(bf16),
                  We3.astype(bf16), row(be3).astype(bf16),
                  W_ag, W_blk, b_blk,
                  Wa3.astype(bf16), row(ba3),
                  Wc1[:Cout], Wc1[Cout:], row(bc1), Wc2, row(bc2))

    out = pl.pallas_call(
        functools.partial(_edge_kernel, ti=TI, n=N, cout=Cout),
        grid=(B, N // TI),
        in_specs=[
            pl.BlockSpec((1, TI, E, N), lambda b, i: (b, i, 0, 0)),
            pl.BlockSpec((1, TI, N), lambda b, i: (b, i, 0)),
            pl.BlockSpec((TI * N, TI), lambda b, i: (0, 0)),
            pl.BlockSpec((1, TI, 128), lambda b, i: (b, i, 0)),
            pl.BlockSpec((1, N, 128), lambda b, i: (b, 0, 0)),
            pl.BlockSpec((1, N, Cout), lambda b, i: (b, 0, 0)),
            pl.BlockSpec((1, TI, Cout), lambda b, i: (b, i, 0)),
        ] + [wspec(a) for a in weight_ins],
        out_specs=pl.BlockSpec((1, TI, Cout), lambda b, i: (b, i, 0)),
        out_shape=jax.ShapeDtypeStruct((B, N, Cout), f32),
        compiler_params=pltpu.CompilerParams(
            dimension_semantics=("parallel", "parallel")),
    )(efT, adjacency, ind, addi, addj.astype(bf16),
      t.astype(bf16), sf, *weight_ins)
    return out
